# baseline (speedup 1.0000x reference)
import functools
import math

import jax
import jax.numpy as jnp
from jax import lax
from jax.experimental import pallas as pl
from jax.experimental.pallas import tpu as pltpu
from jax.experimental.pallas import tpu_sc as plsc

NEG_INF = -1e30
TILE = 256


def _route_body(x_ref, wg_ref, bg_ref, dest_ref, gates_ref, meta_ref,
                *, num_experts, tile, g_max):
    T = x_ref.shape[0]
    P = 2 * T
    scores = jnp.dot(x_ref[...], wg_ref[...],
                     preferred_element_type=jnp.float32) + bg_ref[...]
    iota = jax.lax.broadcasted_iota(jnp.int32, scores.shape, 1)
    m0 = jnp.max(scores, axis=-1, keepdims=True)
    i0 = jnp.min(jnp.where(scores == m0, iota, num_experts),
                 axis=-1, keepdims=True)
    masked = jnp.where(iota == i0, NEG_INF, scores)
    m1 = jnp.max(masked, axis=-1, keepdims=True)
    i1 = jnp.min(jnp.where(masked == m1, iota, num_experts),
                 axis=-1, keepdims=True)
    g0 = 1.0 / (1.0 + jnp.exp(m1 - m0))

    onehot = jnp.concatenate(
        [(iota == i0).astype(jnp.float32), (iota == i1).astype(jnp.float32)],
        axis=0)

    incl = onehot
    s = 1
    while s < P:
        shifted = jnp.concatenate(
            [jnp.zeros((s, num_experts), jnp.float32), incl[:P - s]], axis=0)
        incl = incl + shifted
        s *= 2
    strict = incl - onehot
    counts = incl[P - 1:P, :]

    counts_i = counts.astype(jnp.int32)
    padded = ((counts_i + (tile - 1)) // tile) * tile
    padded_f = padded.astype(jnp.float32)
    er = jax.lax.broadcasted_iota(jnp.int32, (num_experts, num_experts), 0)
    ec = jax.lax.broadcasted_iota(jnp.int32, (num_experts, num_experts), 1)
    upper = (er < ec).astype(jnp.float32)
    po = jnp.dot(padded_f, upper, preferred_element_type=jnp.float32)
    off_next = po + padded_f

    dest = jnp.sum(onehot * (po + strict), axis=-1, keepdims=True)
    dest_ref[...] = dest.astype(jnp.int32)
    gates_ref[...] = jnp.concatenate([g0, 1.0 - g0], axis=0)

    gi = (jax.lax.broadcasted_iota(jnp.int32, (g_max, num_experts), 0)
          * tile).astype(jnp.float32)
    et = jnp.sum((gi >= off_next).astype(jnp.float32), axis=-1, keepdims=True)
    et = jnp.minimum(et.astype(jnp.int32), num_experts - 1)
    ntiles = (off_next[:, num_experts - 1:] / tile).astype(jnp.int32)
    meta_ref[...] = jnp.concatenate([et, ntiles], axis=0)


def _make_dispatch(T, EMB, NPAD):
    NW = 32
    TPW = T // NW
    CH = 32
    NCH = TPW // CH
    NCHUNK = 2 * NCH
    mesh = plsc.VectorSubcoreMesh(core_axis_name="c", subcore_axis_name="s")

    EMB2 = EMB // 2
    @functools.partial(
        pl.kernel, mesh=mesh,
        out_type=[jax.ShapeDtypeStruct((NPAD, EMB2), jnp.int32),
                  jax.ShapeDtypeStruct((NPAD,), jnp.float32)],
        scratch_types=[pltpu.VMEM((2, NCH, CH), jnp.int32),
                       pltpu.VMEM((2, NCH, CH), jnp.float32),
                       pltpu.VMEM((CH, EMB2), jnp.int32),
                       pltpu.VMEM((CH, EMB2), jnp.int32),
                       pltpu.SemaphoreType.DMA,
                       pltpu.SemaphoreType.DMA,
                       pltpu.SemaphoreType.DMA,
                       pltpu.SemaphoreType.DMA,
                       pltpu.SemaphoreType.DMA,
                       pltpu.SemaphoreType.DMA],
    )
    def dispatch(x_hbm, dest_hbm, gates_hbm, xg_hbm, rg_hbm,
                 idx_all, g_all, rows_a, rows_b,
                 gs_a, gs_b, ss_a, ss_b, sg_a, sg_b):
        wid = lax.axis_index("s") * 2 + lax.axis_index("c")
        base = wid * TPW
        pltpu.sync_copy(dest_hbm.at[0, wid], idx_all.at[0])
        pltpu.sync_copy(dest_hbm.at[1, wid], idx_all.at[1])
        pltpu.sync_copy(gates_hbm.at[0, wid], g_all.at[0])
        pltpu.sync_copy(gates_hbm.at[1, wid], g_all.at[1])

        bufs = (rows_a, rows_b)
        gsems = (gs_a, gs_b)
        ssems = (ss_a, ss_b)
        gtsems = (sg_a, sg_b)
        chunks = [(k, c) for k in range(2) for c in range(NCH)]

        def gather_in(n):
            _, c = chunks[n]
            tok = base + c * CH
            return pltpu.async_copy(x_hbm.at[pl.ds(tok, CH)],
                                    bufs[n % 2], gsems[n % 2])

        cp_in = {0: gather_in(0), 1: gather_in(1)}
        cp_out = {}
        cp_gt = {}
        for n in range(NCHUNK):
            b = n % 2
            k, c = chunks[n]
            cp_in[n].wait()
            cp_out[n] = pltpu.async_copy(
                bufs[b], xg_hbm.at[idx_all.at[k, c]], ssems[b])
            cp_gt[n] = pltpu.async_copy(
                g_all.at[k, c], rg_hbm.at[idx_all.at[k, c]], gtsems[b])
            if n + 2 < NCHUNK:
                cp_out[n].wait()
                cp_gt[n].wait()
                cp_in[n + 2] = gather_in(n + 2)
        cp_out[NCHUNK - 2].wait()
        cp_gt[NCHUNK - 2].wait()
        cp_out[NCHUNK - 1].wait()
        cp_gt[NCHUNK - 1].wait()

    return dispatch


def _ffn_body(meta_ref, xg_ref, rg_ref, w1_ref, b1_ref, w2_ref, b2_ref,
              yg_ref, *, g_max):
    g = pl.program_id(0)

    @pl.when(g < meta_ref[g_max])
    def _compute():
        h = jnp.dot(xg_ref[...].astype(jnp.float32), w1_ref[0],
                    preferred_element_type=jnp.float32) + b1_ref[0]
        h = 0.5 * h * (1.0 + jax.lax.erf(h * (1.0 / math.sqrt(2.0))))
        y = jnp.dot(h, w2_ref[0],
                    preferred_element_type=jnp.float32) + b2_ref[0]
        yg_ref[...] = y * rg_ref[...]


def _make_combine(T, EMB, NPAD):
    NW = 32
    TPW = T // NW
    CH = 16
    NCH = TPW // CH
    SEG = EMB // 16
    UNR = 8
    mesh = plsc.VectorSubcoreMesh(core_axis_name="c", subcore_axis_name="s")

    @functools.partial(
        pl.kernel, mesh=mesh,
        out_type=jax.ShapeDtypeStruct((T, EMB), jnp.float32),
        scratch_types=[pltpu.VMEM((2, NCH, CH), jnp.int32),
                       pltpu.VMEM((CH, EMB), jnp.float32),
                       pltpu.VMEM((CH, EMB), jnp.float32),
                       pltpu.VMEM((CH, EMB), jnp.float32),
                       pltpu.VMEM((CH, EMB), jnp.float32),
                       pltpu.SemaphoreType.DMA,
                       pltpu.SemaphoreType.DMA,
                       pltpu.SemaphoreType.DMA,
                       pltpu.SemaphoreType.DMA,
                       pltpu.SemaphoreType.DMA,
                       pltpu.SemaphoreType.DMA],
    )
    def combine(yg_hbm, dest_hbm, out_hbm,
                idx_all, r0_a, r1_a, r0_b, r1_b,
                s0_a, s1_a, s0_b, s1_b, so_a, so_b):
        wid = lax.axis_index("s") * 2 + lax.axis_index("c")
        base = wid * TPW
        pltpu.sync_copy(dest_hbm.at[0, wid], idx_all.at[0])
        pltpu.sync_copy(dest_hbm.at[1, wid], idx_all.at[1])

        r0s = (r0_a, r0_b)
        r1s = (r1_a, r1_b)
        s0s = (s0_a, s0_b)
        s1s = (s1_a, s1_b)
        sos = (so_a, so_b)

        def gathers(n):
            b = n % 2
            return (pltpu.async_copy(yg_hbm.at[idx_all.at[0, n]],
                                     r0s[b], s0s[b]),
                    pltpu.async_copy(yg_hbm.at[idx_all.at[1, n]],
                                     r1s[b], s1s[b]))

        cp_g = {0: gathers(0), 1: gathers(1)}
        cp_o = {}
        for n in range(NCH):
            b = n % 2
            cp_g[n][0].wait()
            cp_g[n][1].wait()
            r0_v, r1_v = r0s[b], r1s[b]

            def add_body(q, carry, r0_v=r0_v, r1_v=r1_v):
                row = q // (SEG // UNR)
                blk = (q % (SEG // UNR)) * UNR * 16
                for u in range(UNR):
                    j = blk + u * 16
                    r0_v[row, pl.ds(j, 16)] = (r0_v[row, pl.ds(j, 16)]
                                               + r1_v[row, pl.ds(j, 16)])
                return carry

            lax.fori_loop(0, CH * SEG // UNR, add_body, 0)
            tok = base + n * CH
            cp_o[n] = pltpu.async_copy(r0_v, out_hbm.at[pl.ds(tok, CH)],
                                       sos[b])
            if n + 2 < NCH:
                cp_o[n].wait()
                cp_g[n + 2] = gathers(n + 2)
        cp_o[NCH - 2].wait()
        cp_o[NCH - 1].wait()

    return combine


def kernel(x, Wg, bg, W1, b1, W2, b2):
    B, N, EMB = x.shape
    T = B * N
    E, _, HID = W1.shape
    P = 2 * T
    G_MAX = P // TILE + E
    NPAD = G_MAX * TILE
    xf = x.reshape(T, EMB)

    dest, gates, meta = pl.pallas_call(
        functools.partial(_route_body, num_experts=E, tile=TILE, g_max=G_MAX),
        in_specs=[
            pl.BlockSpec((T, EMB), lambda: (0, 0)),
            pl.BlockSpec((EMB, E), lambda: (0, 0)),
            pl.BlockSpec((1, E), lambda: (0, 0)),
        ],
        out_specs=[pl.BlockSpec((P, 1), lambda: (0, 0)),
                   pl.BlockSpec((P, 1), lambda: (0, 0)),
                   pl.BlockSpec((G_MAX + 1, 1), lambda: (0, 0))],
        out_shape=[jax.ShapeDtypeStruct((P, 1), jnp.int32),
                   jax.ShapeDtypeStruct((P, 1), jnp.float32),
                   jax.ShapeDtypeStruct((G_MAX + 1, 1), jnp.int32)],
    )(xf, Wg, bg.reshape(1, E))

    NW = 32
    meta_flat = meta.reshape(G_MAX + 1)
    dest_d = dest.reshape(2, NW, 4, 32)
    gates_d = gates.reshape(2, NW, 4, 32)
    dest_c = dest.reshape(2, NW, 8, 16)

    xi = jax.lax.bitcast_convert_type(
        xf.astype(jnp.bfloat16).reshape(T, EMB // 2, 2), jnp.int32)
    xg_i, rg = _make_dispatch(T, EMB, NPAD)(xi, dest_d, gates_d)
    xg = jax.lax.bitcast_convert_type(xg_i, jnp.bfloat16).reshape(NPAD, EMB)

    yg = pl.pallas_call(
        functools.partial(_ffn_body, g_max=G_MAX),
        grid_spec=pltpu.PrefetchScalarGridSpec(
            num_scalar_prefetch=1,
            grid=(G_MAX,),
            in_specs=[
                pl.BlockSpec((TILE, EMB), lambda g, m: (g, 0)),
                pl.BlockSpec((TILE, 1), lambda g, m: (g, 0)),
                pl.BlockSpec((1, EMB, HID), lambda g, m: (m[g], 0, 0)),
                pl.BlockSpec((1, 1, HID), lambda g, m: (m[g], 0, 0)),
                pl.BlockSpec((1, HID, EMB), lambda g, m: (m[g], 0, 0)),
                pl.BlockSpec((1, 1, EMB), lambda g, m: (m[g], 0, 0)),
            ],
            out_specs=pl.BlockSpec((TILE, EMB), lambda g, m: (g, 0)),
        ),
        out_shape=jax.ShapeDtypeStruct((NPAD, EMB), jnp.float32),
    )(meta_flat, xg, rg.reshape(NPAD, 1), W1, b1.reshape(E, 1, HID),
      W2, b2.reshape(E, 1, EMB))

    out = _make_combine(T, EMB, NPAD)(yg, dest_c)
    return out.reshape(B, N, EMB)


# speedup vs baseline: 2.2608x; 2.2608x over previous
import functools
import math

import jax
import jax.numpy as jnp
from jax import lax
from jax.experimental import pallas as pl
from jax.experimental.pallas import tpu as pltpu
from jax.experimental.pallas import tpu_sc as plsc

NEG_INF = -1e30
TILE = 256


def _route_body(x_ref, wg_ref, bg_ref, dest_ref, gates_ref, meta_ref,
                *, num_experts, tile, g_max):
    T = x_ref.shape[0]
    P = 2 * T
    scores = jnp.dot(x_ref[...], wg_ref[...],
                     preferred_element_type=jnp.float32) + bg_ref[...]
    iota = jax.lax.broadcasted_iota(jnp.int32, scores.shape, 1)
    m0 = jnp.max(scores, axis=-1, keepdims=True)
    i0 = jnp.min(jnp.where(scores == m0, iota, num_experts),
                 axis=-1, keepdims=True)
    masked = jnp.where(iota == i0, NEG_INF, scores)
    m1 = jnp.max(masked, axis=-1, keepdims=True)
    i1 = jnp.min(jnp.where(masked == m1, iota, num_experts),
                 axis=-1, keepdims=True)
    g0 = 1.0 / (1.0 + jnp.exp(m1 - m0))

    onehot = jnp.concatenate(
        [(iota == i0).astype(jnp.float32), (iota == i1).astype(jnp.float32)],
        axis=0)

    incl = onehot
    s = 1
    while s < P:
        shifted = jnp.concatenate(
            [jnp.zeros((s, num_experts), jnp.float32), incl[:P - s]], axis=0)
        incl = incl + shifted
        s *= 2
    strict = incl - onehot
    counts = incl[P - 1:P, :]

    counts_i = counts.astype(jnp.int32)
    padded = ((counts_i + (tile - 1)) // tile) * tile
    padded_f = padded.astype(jnp.float32)
    er = jax.lax.broadcasted_iota(jnp.int32, (num_experts, num_experts), 0)
    ec = jax.lax.broadcasted_iota(jnp.int32, (num_experts, num_experts), 1)
    upper = (er < ec).astype(jnp.float32)
    po = jnp.dot(padded_f, upper, preferred_element_type=jnp.float32)
    off_next = po + padded_f

    dest = jnp.sum(onehot * (po + strict), axis=-1, keepdims=True)
    dest_ref[...] = dest.astype(jnp.int32)
    gates_ref[...] = jnp.concatenate([g0, 1.0 - g0], axis=0)

    gi = (jax.lax.broadcasted_iota(jnp.int32, (g_max, num_experts), 0)
          * tile).astype(jnp.float32)
    et = jnp.sum((gi >= off_next).astype(jnp.float32), axis=-1, keepdims=True)
    et = jnp.minimum(et.astype(jnp.int32), num_experts - 1)
    ntiles = (off_next[:, num_experts - 1:] / tile).astype(jnp.int32)
    meta_ref[...] = jnp.concatenate([et, ntiles], axis=0)


def _make_dispatch(T, EMB, NPAD):
    NW = 32
    TPW = T // NW
    CH = 32
    NCH = TPW // CH
    NCHUNK = 2 * NCH
    mesh = plsc.VectorSubcoreMesh(core_axis_name="c", subcore_axis_name="s")

    @functools.partial(
        pl.kernel, mesh=mesh,
        out_type=[jax.ShapeDtypeStruct((NPAD, EMB), jnp.float32),
                  jax.ShapeDtypeStruct((NPAD,), jnp.float32)],
        scratch_types=[pltpu.VMEM((2, NCH, CH), jnp.int32),
                       pltpu.VMEM((2, NCH, CH), jnp.float32),
                       pltpu.VMEM((CH, EMB), jnp.float32),
                       pltpu.VMEM((CH, EMB), jnp.float32),
                       pltpu.SemaphoreType.DMA,
                       pltpu.SemaphoreType.DMA,
                       pltpu.SemaphoreType.DMA,
                       pltpu.SemaphoreType.DMA,
                       pltpu.SemaphoreType.DMA,
                       pltpu.SemaphoreType.DMA],
    )
    def dispatch(x_hbm, dest_hbm, gates_hbm, xg_hbm, rg_hbm,
                 idx_all, g_all, rows_a, rows_b,
                 gs_a, gs_b, ss_a, ss_b, sg_a, sg_b):
        wid = lax.axis_index("s") * 2 + lax.axis_index("c")
        base = wid * TPW
        pltpu.sync_copy(dest_hbm.at[0, wid], idx_all.at[0])
        pltpu.sync_copy(dest_hbm.at[1, wid], idx_all.at[1])
        pltpu.sync_copy(gates_hbm.at[0, wid], g_all.at[0])
        pltpu.sync_copy(gates_hbm.at[1, wid], g_all.at[1])

        bufs = (rows_a, rows_b)
        gsems = (gs_a, gs_b)
        ssems = (ss_a, ss_b)
        gtsems = (sg_a, sg_b)
        chunks = [(k, c) for k in range(2) for c in range(NCH)]

        def gather_in(n):
            _, c = chunks[n]
            tok = base + c * CH
            return pltpu.async_copy(x_hbm.at[pl.ds(tok, CH)],
                                    bufs[n % 2], gsems[n % 2])

        cp_in = {0: gather_in(0), 1: gather_in(1)}
        cp_out = {}
        cp_gt = {}
        for n in range(NCHUNK):
            b = n % 2
            k, c = chunks[n]
            cp_in[n].wait()
            cp_out[n] = pltpu.async_copy(
                bufs[b], xg_hbm.at[idx_all.at[k, c]], ssems[b])
            cp_gt[n] = pltpu.async_copy(
                g_all.at[k, c], rg_hbm.at[idx_all.at[k, c]], gtsems[b])
            if n + 2 < NCHUNK:
                cp_out[n].wait()
                cp_gt[n].wait()
                cp_in[n + 2] = gather_in(n + 2)
        cp_out[NCHUNK - 2].wait()
        cp_gt[NCHUNK - 2].wait()
        cp_out[NCHUNK - 1].wait()
        cp_gt[NCHUNK - 1].wait()

    return dispatch


def _ffn_body(meta_ref, xg_ref, rg_ref, w1_ref, b1_ref, w2_ref, b2_ref,
              yg_ref, *, g_max):
    g = pl.program_id(0)

    @pl.when(g < meta_ref[g_max])
    def _compute():
        h = jnp.dot(xg_ref[...], w1_ref[0],
                    preferred_element_type=jnp.float32) + b1_ref[0]
        h = 0.5 * h * (1.0 + jax.lax.erf(h * (1.0 / math.sqrt(2.0))))
        y = jnp.dot(h, w2_ref[0],
                    preferred_element_type=jnp.float32) + b2_ref[0]
        yg_ref[...] = y * rg_ref[...]


def _make_combine(T, EMB, NPAD):
    NW = 32
    TPW = T // NW
    CH = 16
    NCH = TPW // CH
    SEG = EMB // 16
    UNR = 8
    mesh = plsc.VectorSubcoreMesh(core_axis_name="c", subcore_axis_name="s")

    @functools.partial(
        pl.kernel, mesh=mesh,
        out_type=jax.ShapeDtypeStruct((T, EMB), jnp.float32),
        scratch_types=[pltpu.VMEM((2, NCH, CH), jnp.int32),
                       pltpu.VMEM((CH, EMB), jnp.float32),
                       pltpu.VMEM((CH, EMB), jnp.float32),
                       pltpu.VMEM((CH, EMB), jnp.float32),
                       pltpu.VMEM((CH, EMB), jnp.float32),
                       pltpu.SemaphoreType.DMA,
                       pltpu.SemaphoreType.DMA,
                       pltpu.SemaphoreType.DMA,
                       pltpu.SemaphoreType.DMA,
                       pltpu.SemaphoreType.DMA,
                       pltpu.SemaphoreType.DMA],
    )
    def combine(yg_hbm, dest_hbm, out_hbm,
                idx_all, r0_a, r1_a, r0_b, r1_b,
                s0_a, s1_a, s0_b, s1_b, so_a, so_b):
        wid = lax.axis_index("s") * 2 + lax.axis_index("c")
        base = wid * TPW
        pltpu.sync_copy(dest_hbm.at[0, wid], idx_all.at[0])
        pltpu.sync_copy(dest_hbm.at[1, wid], idx_all.at[1])

        r0s = (r0_a, r0_b)
        r1s = (r1_a, r1_b)
        s0s = (s0_a, s0_b)
        s1s = (s1_a, s1_b)
        sos = (so_a, so_b)

        def gathers(n):
            b = n % 2
            return (pltpu.async_copy(yg_hbm.at[idx_all.at[0, n]],
                                     r0s[b], s0s[b]),
                    pltpu.async_copy(yg_hbm.at[idx_all.at[1, n]],
                                     r1s[b], s1s[b]))

        cp_g = {0: gathers(0), 1: gathers(1)}
        cp_o = {}
        for n in range(NCH):
            b = n % 2
            cp_g[n][0].wait()
            cp_g[n][1].wait()
            r0_v, r1_v = r0s[b], r1s[b]

            def add_body(q, carry, r0_v=r0_v, r1_v=r1_v):
                row = q // (SEG // UNR)
                blk = (q % (SEG // UNR)) * UNR * 16
                for u in range(UNR):
                    j = blk + u * 16
                    r0_v[row, pl.ds(j, 16)] = (r0_v[row, pl.ds(j, 16)]
                                               + r1_v[row, pl.ds(j, 16)])
                return carry

            lax.fori_loop(0, CH * SEG // UNR, add_body, 0)
            tok = base + n * CH
            cp_o[n] = pltpu.async_copy(r0_v, out_hbm.at[pl.ds(tok, CH)],
                                       sos[b])
            if n + 2 < NCH:
                cp_o[n].wait()
                cp_g[n + 2] = gathers(n + 2)
        cp_o[NCH - 2].wait()
        cp_o[NCH - 1].wait()

    return combine


def kernel(x, Wg, bg, W1, b1, W2, b2):
    B, N, EMB = x.shape
    T = B * N
    E, _, HID = W1.shape
    P = 2 * T
    G_MAX = P // TILE + E
    NPAD = G_MAX * TILE
    xf = x.reshape(T, EMB)

    dest, gates, meta = pl.pallas_call(
        functools.partial(_route_body, num_experts=E, tile=TILE, g_max=G_MAX),
        in_specs=[
            pl.BlockSpec((T, EMB), lambda: (0, 0)),
            pl.BlockSpec((EMB, E), lambda: (0, 0)),
            pl.BlockSpec((1, E), lambda: (0, 0)),
        ],
        out_specs=[pl.BlockSpec((P, 1), lambda: (0, 0)),
                   pl.BlockSpec((P, 1), lambda: (0, 0)),
                   pl.BlockSpec((G_MAX + 1, 1), lambda: (0, 0))],
        out_shape=[jax.ShapeDtypeStruct((P, 1), jnp.int32),
                   jax.ShapeDtypeStruct((P, 1), jnp.float32),
                   jax.ShapeDtypeStruct((G_MAX + 1, 1), jnp.int32)],
    )(xf, Wg, bg.reshape(1, E))

    NW = 32
    meta_flat = meta.reshape(G_MAX + 1)
    dest_d = dest.reshape(2, NW, 4, 32)
    gates_d = gates.reshape(2, NW, 4, 32)
    dest_c = dest.reshape(2, NW, 8, 16)

    xg, rg = _make_dispatch(T, EMB, NPAD)(xf, dest_d, gates_d)

    yg = pl.pallas_call(
        functools.partial(_ffn_body, g_max=G_MAX),
        grid_spec=pltpu.PrefetchScalarGridSpec(
            num_scalar_prefetch=1,
            grid=(G_MAX,),
            in_specs=[
                pl.BlockSpec((TILE, EMB), lambda g, m: (g, 0)),
                pl.BlockSpec((TILE, 1), lambda g, m: (g, 0)),
                pl.BlockSpec((1, EMB, HID), lambda g, m: (m[g], 0, 0)),
                pl.BlockSpec((1, 1, HID), lambda g, m: (m[g], 0, 0)),
                pl.BlockSpec((1, HID, EMB), lambda g, m: (m[g], 0, 0)),
                pl.BlockSpec((1, 1, EMB), lambda g, m: (m[g], 0, 0)),
            ],
            out_specs=pl.BlockSpec((TILE, EMB), lambda g, m: (g, 0)),
        ),
        out_shape=jax.ShapeDtypeStruct((NPAD, EMB), jnp.float32),
    )(meta_flat, xg, rg.reshape(NPAD, 1), W1, b1.reshape(E, 1, HID),
      W2, b2.reshape(E, 1, EMB))

    out = _make_combine(T, EMB, NPAD)(yg, dest_c)
    return out.reshape(B, N, EMB)


# speedup vs baseline: 2.3167x; 1.0248x over previous
import functools
import math

import jax
import jax.numpy as jnp
from jax import lax
from jax.experimental import pallas as pl
from jax.experimental.pallas import tpu as pltpu
from jax.experimental.pallas import tpu_sc as plsc

NEG_INF = -1e30
TILE = 256


def _route_body(x_ref, wg_ref, bg_ref, dest_ref, gates_ref, meta_ref,
                *, num_experts, tile, g_max):
    T = x_ref.shape[0]
    P = 2 * T
    scores = jnp.dot(x_ref[...], wg_ref[...],
                     preferred_element_type=jnp.float32) + bg_ref[...]
    iota = jax.lax.broadcasted_iota(jnp.int32, scores.shape, 1)
    m0 = jnp.max(scores, axis=-1, keepdims=True)
    i0 = jnp.min(jnp.where(scores == m0, iota, num_experts),
                 axis=-1, keepdims=True)
    masked = jnp.where(iota == i0, NEG_INF, scores)
    m1 = jnp.max(masked, axis=-1, keepdims=True)
    i1 = jnp.min(jnp.where(masked == m1, iota, num_experts),
                 axis=-1, keepdims=True)
    g0 = 1.0 / (1.0 + jnp.exp(m1 - m0))

    onehot = jnp.concatenate(
        [(iota == i0).astype(jnp.float32), (iota == i1).astype(jnp.float32)],
        axis=0)

    incl = onehot
    s = 1
    while s < P:
        shifted = jnp.concatenate(
            [jnp.zeros((s, num_experts), jnp.float32), incl[:P - s]], axis=0)
        incl = incl + shifted
        s *= 2
    strict = incl - onehot
    counts = incl[P - 1:P, :]

    counts_i = counts.astype(jnp.int32)
    padded = ((counts_i + (tile - 1)) // tile) * tile
    padded_f = padded.astype(jnp.float32)
    er = jax.lax.broadcasted_iota(jnp.int32, (num_experts, num_experts), 0)
    ec = jax.lax.broadcasted_iota(jnp.int32, (num_experts, num_experts), 1)
    upper = (er < ec).astype(jnp.float32)
    po = jnp.dot(padded_f, upper, preferred_element_type=jnp.float32)
    off_next = po + padded_f

    dest = jnp.sum(onehot * (po + strict), axis=-1, keepdims=True)
    dest_ref[...] = dest.astype(jnp.int32)
    gates_ref[...] = jnp.concatenate([g0, 1.0 - g0], axis=0)

    gi = (jax.lax.broadcasted_iota(jnp.int32, (g_max, num_experts), 0)
          * tile).astype(jnp.float32)
    et = jnp.sum((gi >= off_next).astype(jnp.float32), axis=-1, keepdims=True)
    et = jnp.minimum(et.astype(jnp.int32), num_experts - 1)
    ntiles = (off_next[:, num_experts - 1:] / tile).astype(jnp.int32)
    meta_ref[...] = jnp.concatenate([et, ntiles], axis=0)


def _make_dispatch(T, EMB, NPAD):
    NW = 32
    TPW = T // NW
    CH = 16
    NCH = TPW // CH
    NCHUNK = 2 * NCH
    NBUF = 6
    mesh = plsc.VectorSubcoreMesh(core_axis_name="c", subcore_axis_name="s")

    @functools.partial(
        pl.kernel, mesh=mesh,
        out_type=[jax.ShapeDtypeStruct((NPAD, EMB), jnp.float32),
                  jax.ShapeDtypeStruct((NPAD,), jnp.float32)],
        scratch_types=([pltpu.VMEM((2, NCH, CH), jnp.int32),
                        pltpu.VMEM((2, NCH, CH), jnp.float32)]
                       + [pltpu.VMEM((CH, EMB), jnp.float32)] * NBUF
                       + [pltpu.SemaphoreType.DMA] * (2 * NBUF + 1)),
    )
    def dispatch(x_hbm, dest_hbm, gates_hbm, xg_hbm, rg_hbm,
                 idx_all, g_all, *bufs_and_sems):
        bufs = bufs_and_sems[:NBUF]
        gsems = bufs_and_sems[NBUF:2 * NBUF]
        ssems = bufs_and_sems[2 * NBUF:3 * NBUF]
        sg = bufs_and_sems[3 * NBUF]
        wid = lax.axis_index("s") * 2 + lax.axis_index("c")
        base = wid * TPW
        pltpu.sync_copy(dest_hbm.at[0, wid], idx_all.at[0])
        pltpu.sync_copy(dest_hbm.at[1, wid], idx_all.at[1])
        pltpu.sync_copy(gates_hbm.at[0, wid], g_all.at[0])
        pltpu.sync_copy(gates_hbm.at[1, wid], g_all.at[1])

        chunks = [(k, c) for k in range(2) for c in range(NCH)]

        def gather_in(n):
            _, c = chunks[n]
            tok = base + c * CH
            return pltpu.async_copy(x_hbm.at[pl.ds(tok, CH)],
                                    bufs[n % NBUF], gsems[n % NBUF])

        cp_in = {n: gather_in(n) for n in range(min(NBUF, NCHUNK))}
        cp_out = {}
        cp_gt = {}
        waited = set()
        for n in range(NCHUNK):
            b = n % NBUF
            k, c = chunks[n]
            cp_in[n].wait()
            cp_out[n] = pltpu.async_copy(
                bufs[b], xg_hbm.at[idx_all.at[k, c]], ssems[b])
            cp_gt[n] = pltpu.async_copy(
                g_all.at[k, c], rg_hbm.at[idx_all.at[k, c]], sg)
            m = n - 2
            if m >= 0 and m + NBUF < NCHUNK:
                cp_out[m].wait()
                waited.add(m)
                cp_in[m + NBUF] = gather_in(m + NBUF)
        for n in range(NCHUNK):
            if n not in waited:
                cp_out[n].wait()
            cp_gt[n].wait()

    return dispatch


def _ffn_body(meta_ref, xg_ref, rg_ref, w1_ref, b1_ref, w2_ref, b2_ref,
              yg_ref, *, g_max):
    g = pl.program_id(0)

    @pl.when(g < meta_ref[g_max])
    def _compute():
        h = jnp.dot(xg_ref[...], w1_ref[0],
                    preferred_element_type=jnp.float32) + b1_ref[0]
        h = 0.5 * h * (1.0 + jax.lax.erf(h * (1.0 / math.sqrt(2.0))))
        y = jnp.dot(h, w2_ref[0],
                    preferred_element_type=jnp.float32) + b2_ref[0]
        yg_ref[...] = y * rg_ref[...]


def _make_combine(T, EMB, NPAD):
    NW = 32
    TPW = T // NW
    CH = 16
    NCH = TPW // CH
    SEG = EMB // 16
    UNR = 8
    mesh = plsc.VectorSubcoreMesh(core_axis_name="c", subcore_axis_name="s")

    @functools.partial(
        pl.kernel, mesh=mesh,
        out_type=jax.ShapeDtypeStruct((T, EMB), jnp.float32),
        scratch_types=[pltpu.VMEM((2, NCH, CH), jnp.int32),
                       pltpu.VMEM((CH, EMB), jnp.float32),
                       pltpu.VMEM((CH, EMB), jnp.float32),
                       pltpu.VMEM((CH, EMB), jnp.float32),
                       pltpu.VMEM((CH, EMB), jnp.float32),
                       pltpu.SemaphoreType.DMA,
                       pltpu.SemaphoreType.DMA,
                       pltpu.SemaphoreType.DMA,
                       pltpu.SemaphoreType.DMA,
                       pltpu.SemaphoreType.DMA,
                       pltpu.SemaphoreType.DMA],
    )
    def combine(yg_hbm, dest_hbm, out_hbm,
                idx_all, r0_a, r1_a, r0_b, r1_b,
                s0_a, s1_a, s0_b, s1_b, so_a, so_b):
        wid = lax.axis_index("s") * 2 + lax.axis_index("c")
        base = wid * TPW
        pltpu.sync_copy(dest_hbm.at[0, wid], idx_all.at[0])
        pltpu.sync_copy(dest_hbm.at[1, wid], idx_all.at[1])

        r0s = (r0_a, r0_b)
        r1s = (r1_a, r1_b)
        s0s = (s0_a, s0_b)
        s1s = (s1_a, s1_b)
        sos = (so_a, so_b)

        def gathers(n):
            b = n % 2
            return (pltpu.async_copy(yg_hbm.at[idx_all.at[0, n]],
                                     r0s[b], s0s[b]),
                    pltpu.async_copy(yg_hbm.at[idx_all.at[1, n]],
                                     r1s[b], s1s[b]))

        cp_g = {0: gathers(0), 1: gathers(1)}
        cp_o = {}
        for n in range(NCH):
            b = n % 2
            cp_g[n][0].wait()
            cp_g[n][1].wait()
            r0_v, r1_v = r0s[b], r1s[b]

            def add_body(q, carry, r0_v=r0_v, r1_v=r1_v):
                row = q // (SEG // UNR)
                blk = (q % (SEG // UNR)) * UNR * 16
                for u in range(UNR):
                    j = blk + u * 16
                    r0_v[row, pl.ds(j, 16)] = (r0_v[row, pl.ds(j, 16)]
                                               + r1_v[row, pl.ds(j, 16)])
                return carry

            lax.fori_loop(0, CH * SEG // UNR, add_body, 0)
            tok = base + n * CH
            cp_o[n] = pltpu.async_copy(r0_v, out_hbm.at[pl.ds(tok, CH)],
                                       sos[b])
            if n + 2 < NCH:
                cp_o[n].wait()
                cp_g[n + 2] = gathers(n + 2)
        cp_o[NCH - 2].wait()
        cp_o[NCH - 1].wait()

    return combine


def kernel(x, Wg, bg, W1, b1, W2, b2):
    B, N, EMB = x.shape
    T = B * N
    E, _, HID = W1.shape
    P = 2 * T
    G_MAX = P // TILE + E
    NPAD = G_MAX * TILE
    xf = x.reshape(T, EMB)

    dest, gates, meta = pl.pallas_call(
        functools.partial(_route_body, num_experts=E, tile=TILE, g_max=G_MAX),
        in_specs=[
            pl.BlockSpec((T, EMB), lambda: (0, 0)),
            pl.BlockSpec((EMB, E), lambda: (0, 0)),
            pl.BlockSpec((1, E), lambda: (0, 0)),
        ],
        out_specs=[pl.BlockSpec((P, 1), lambda: (0, 0)),
                   pl.BlockSpec((P, 1), lambda: (0, 0)),
                   pl.BlockSpec((G_MAX + 1, 1), lambda: (0, 0))],
        out_shape=[jax.ShapeDtypeStruct((P, 1), jnp.int32),
                   jax.ShapeDtypeStruct((P, 1), jnp.float32),
                   jax.ShapeDtypeStruct((G_MAX + 1, 1), jnp.int32)],
    )(xf, Wg, bg.reshape(1, E))

    NW = 32
    meta_flat = meta.reshape(G_MAX + 1)
    dest_c = dest.reshape(2, NW, 8, 16)
    gates_d = gates.reshape(2, NW, 8, 16)

    xg, rg = _make_dispatch(T, EMB, NPAD)(xf, dest_c, gates_d)

    yg = pl.pallas_call(
        functools.partial(_ffn_body, g_max=G_MAX),
        grid_spec=pltpu.PrefetchScalarGridSpec(
            num_scalar_prefetch=1,
            grid=(G_MAX,),
            in_specs=[
                pl.BlockSpec((TILE, EMB), lambda g, m: (g, 0)),
                pl.BlockSpec((TILE, 1), lambda g, m: (g, 0)),
                pl.BlockSpec((1, EMB, HID), lambda g, m: (m[g], 0, 0)),
                pl.BlockSpec((1, 1, HID), lambda g, m: (m[g], 0, 0)),
                pl.BlockSpec((1, HID, EMB), lambda g, m: (m[g], 0, 0)),
                pl.BlockSpec((1, 1, EMB), lambda g, m: (m[g], 0, 0)),
            ],
            out_specs=pl.BlockSpec((TILE, EMB), lambda g, m: (g, 0)),
        ),
        out_shape=jax.ShapeDtypeStruct((NPAD, EMB), jnp.float32),
    )(meta_flat, xg, rg.reshape(NPAD, 1), W1, b1.reshape(E, 1, HID),
      W2, b2.reshape(E, 1, EMB))

    out = _make_combine(T, EMB, NPAD)(yg, dest_c)
    return out.reshape(B, N, EMB)


# speedup vs baseline: 2.4842x; 1.0723x over previous
import functools
import math

import jax
import jax.numpy as jnp
from jax import lax
from jax.experimental import pallas as pl
from jax.experimental.pallas import tpu as pltpu
from jax.experimental.pallas import tpu_sc as plsc

NEG_INF = -1e30
TILE = 256


def _route_body(x_ref, wg_ref, bg_ref, dest_ref, g0r_ref, g1r_ref, meta_ref,
                *, num_experts, tile, g_max):
    T = x_ref.shape[0]
    P = 2 * T
    scores = jnp.dot(x_ref[...], wg_ref[...],
                     preferred_element_type=jnp.float32) + bg_ref[...]
    iota = jax.lax.broadcasted_iota(jnp.int32, scores.shape, 1)
    m0 = jnp.max(scores, axis=-1, keepdims=True)
    i0 = jnp.min(jnp.where(scores == m0, iota, num_experts),
                 axis=-1, keepdims=True)
    masked = jnp.where(iota == i0, NEG_INF, scores)
    m1 = jnp.max(masked, axis=-1, keepdims=True)
    i1 = jnp.min(jnp.where(masked == m1, iota, num_experts),
                 axis=-1, keepdims=True)
    g0 = 1.0 / (1.0 + jnp.exp(m1 - m0))

    onehot = jnp.concatenate(
        [(iota == i0).astype(jnp.float32), (iota == i1).astype(jnp.float32)],
        axis=0)

    incl = onehot
    s = 1
    while s < P:
        shifted = jnp.concatenate(
            [jnp.zeros((s, num_experts), jnp.float32), incl[:P - s]], axis=0)
        incl = incl + shifted
        s *= 2
    strict = incl - onehot
    counts = incl[P - 1:P, :]

    counts_i = counts.astype(jnp.int32)
    padded = ((counts_i + (tile - 1)) // tile) * tile
    padded_f = padded.astype(jnp.float32)
    er = jax.lax.broadcasted_iota(jnp.int32, (num_experts, num_experts), 0)
    ec = jax.lax.broadcasted_iota(jnp.int32, (num_experts, num_experts), 1)
    upper = (er < ec).astype(jnp.float32)
    po = jnp.dot(padded_f, upper, preferred_element_type=jnp.float32)
    off_next = po + padded_f

    dest = jnp.sum(onehot * (po + strict), axis=-1, keepdims=True)
    dest_ref[...] = dest.astype(jnp.int32)
    lanes = jnp.ones((1, 16), jnp.float32)
    g0r_ref[...] = g0 * lanes
    g1r_ref[...] = (1.0 - g0) * lanes

    gi = (jax.lax.broadcasted_iota(jnp.int32, (g_max, num_experts), 0)
          * tile).astype(jnp.float32)
    et = jnp.sum((gi >= off_next).astype(jnp.float32), axis=-1, keepdims=True)
    et = jnp.minimum(et.astype(jnp.int32), num_experts - 1)
    ntiles = (off_next[:, num_experts - 1:] / tile).astype(jnp.int32)
    meta_ref[...] = jnp.concatenate([et, ntiles], axis=0)


def _make_dispatch(T, EMB, NPAD):
    NW = 32
    TPW = T // NW
    CH = 16
    NCH = TPW // CH
    NCHUNK = 2 * NCH
    NBUF = 6
    mesh = plsc.VectorSubcoreMesh(core_axis_name="c", subcore_axis_name="s")

    @functools.partial(
        pl.kernel, mesh=mesh,
        out_type=jax.ShapeDtypeStruct((NPAD, EMB), jnp.float32),
        scratch_types=([pltpu.VMEM((2, NCH, CH), jnp.int32)]
                       + [pltpu.VMEM((CH, EMB), jnp.float32)] * NBUF
                       + [pltpu.SemaphoreType.DMA] * (2 * NBUF)),
    )
    def dispatch(x_hbm, dest_hbm, xg_hbm, idx_all, *bufs_and_sems):
        bufs = bufs_and_sems[:NBUF]
        gsems = bufs_and_sems[NBUF:2 * NBUF]
        ssems = bufs_and_sems[2 * NBUF:3 * NBUF]
        wid = lax.axis_index("s") * 2 + lax.axis_index("c")
        base = wid * TPW
        pltpu.sync_copy(dest_hbm.at[0, wid], idx_all.at[0])
        pltpu.sync_copy(dest_hbm.at[1, wid], idx_all.at[1])

        chunks = [(k, c) for k in range(2) for c in range(NCH)]

        def gather_in(n):
            _, c = chunks[n]
            tok = base + c * CH
            return pltpu.async_copy(x_hbm.at[pl.ds(tok, CH)],
                                    bufs[n % NBUF], gsems[n % NBUF])

        cp_in = {n: gather_in(n) for n in range(min(NBUF, NCHUNK))}
        cp_out = {}
        waited = set()
        for n in range(NCHUNK):
            b = n % NBUF
            k, c = chunks[n]
            cp_in[n].wait()
            cp_out[n] = pltpu.async_copy(
                bufs[b], xg_hbm.at[idx_all.at[k, c]], ssems[b])
            m = n - 2
            if m >= 0 and m + NBUF < NCHUNK:
                cp_out[m].wait()
                waited.add(m)
                cp_in[m + NBUF] = gather_in(m + NBUF)
        for n in range(NCHUNK):
            if n not in waited:
                cp_out[n].wait()

    return dispatch


def _ffn_body(meta_ref, xg_ref, w1_ref, b1_ref, w2_ref, b2_ref,
              yg_ref, *, g_max):
    g = pl.program_id(0)

    @pl.when(g < meta_ref[g_max])
    def _compute():
        h = jnp.dot(xg_ref[...], w1_ref[0],
                    preferred_element_type=jnp.float32) + b1_ref[0]
        h = 0.5 * h * (1.0 + jax.lax.erf(h * (1.0 / math.sqrt(2.0))))
        y = jnp.dot(h, w2_ref[0],
                    preferred_element_type=jnp.float32) + b2_ref[0]
        yg_ref[...] = y


def _make_combine(T, EMB, NPAD):
    NW = 32
    TPW = T // NW
    CH = 16
    NCH = TPW // CH
    SEG = EMB // 16
    UNR = 8
    mesh = plsc.VectorSubcoreMesh(core_axis_name="c", subcore_axis_name="s")

    @functools.partial(
        pl.kernel, mesh=mesh,
        out_type=jax.ShapeDtypeStruct((T, EMB), jnp.float32),
        scratch_types=[pltpu.VMEM((2, NCH, CH), jnp.int32),
                       pltpu.VMEM((CH, 16), jnp.float32),
                       pltpu.VMEM((CH, 16), jnp.float32),
                       pltpu.VMEM((CH, EMB), jnp.float32),
                       pltpu.VMEM((CH, EMB), jnp.float32),
                       pltpu.VMEM((CH, EMB), jnp.float32),
                       pltpu.VMEM((CH, EMB), jnp.float32),
                       pltpu.SemaphoreType.DMA,
                       pltpu.SemaphoreType.DMA,
                       pltpu.SemaphoreType.DMA,
                       pltpu.SemaphoreType.DMA,
                       pltpu.SemaphoreType.DMA,
                       pltpu.SemaphoreType.DMA],
    )
    def combine(yg_hbm, dest_hbm, g0r_hbm, g1r_hbm, out_hbm,
                idx_all, g0_v, g1_v, r0_a, r1_a, r0_b, r1_b,
                s0_a, s1_a, s0_b, s1_b, so_a, so_b):
        wid = lax.axis_index("s") * 2 + lax.axis_index("c")
        base = wid * TPW
        pltpu.sync_copy(dest_hbm.at[0, wid], idx_all.at[0])
        pltpu.sync_copy(dest_hbm.at[1, wid], idx_all.at[1])

        r0s = (r0_a, r0_b)
        r1s = (r1_a, r1_b)
        s0s = (s0_a, s0_b)
        s1s = (s1_a, s1_b)
        sos = (so_a, so_b)

        def gathers(n):
            b = n % 2
            return (pltpu.async_copy(yg_hbm.at[idx_all.at[0, n]],
                                     r0s[b], s0s[b]),
                    pltpu.async_copy(yg_hbm.at[idx_all.at[1, n]],
                                     r1s[b], s1s[b]))

        cp_g = {0: gathers(0), 1: gathers(1)}
        cp_o = {}
        for n in range(NCH):
            b = n % 2
            pltpu.sync_copy(g0r_hbm.at[wid, n], g0_v)
            pltpu.sync_copy(g1r_hbm.at[wid, n], g1_v)
            cp_g[n][0].wait()
            cp_g[n][1].wait()
            r0_v, r1_v = r0s[b], r1s[b]
            for i in range(CH):
                ga = g0_v[i]
                gb = g1_v[i]

                def add_body(q, carry, i=i, ga=ga, gb=gb,
                             r0_v=r0_v, r1_v=r1_v):
                    blk = q * (UNR * 16)
                    for u in range(UNR):
                        j = blk + u * 16
                        r0_v[i, pl.ds(j, 16)] = (
                            ga * r0_v[i, pl.ds(j, 16)]
                            + gb * r1_v[i, pl.ds(j, 16)])
                    return carry

                lax.fori_loop(0, SEG // UNR, add_body, 0)
            tok = base + n * CH
            cp_o[n] = pltpu.async_copy(r0_v, out_hbm.at[pl.ds(tok, CH)],
                                       sos[b])
            if n + 2 < NCH:
                cp_o[n].wait()
                cp_g[n + 2] = gathers(n + 2)
        cp_o[NCH - 2].wait()
        cp_o[NCH - 1].wait()

    return combine


def kernel(x, Wg, bg, W1, b1, W2, b2):
    B, N, EMB = x.shape
    T = B * N
    E, _, HID = W1.shape
    P = 2 * T
    G_MAX = P // TILE + E
    NPAD = G_MAX * TILE
    xf = x.reshape(T, EMB)

    dest, g0r, g1r, meta = pl.pallas_call(
        functools.partial(_route_body, num_experts=E, tile=TILE, g_max=G_MAX),
        in_specs=[
            pl.BlockSpec((T, EMB), lambda: (0, 0)),
            pl.BlockSpec((EMB, E), lambda: (0, 0)),
            pl.BlockSpec((1, E), lambda: (0, 0)),
        ],
        out_specs=[pl.BlockSpec((P, 1), lambda: (0, 0)),
                   pl.BlockSpec((T, 16), lambda: (0, 0)),
                   pl.BlockSpec((T, 16), lambda: (0, 0)),
                   pl.BlockSpec((G_MAX + 1, 1), lambda: (0, 0))],
        out_shape=[jax.ShapeDtypeStruct((P, 1), jnp.int32),
                   jax.ShapeDtypeStruct((T, 16), jnp.float32),
                   jax.ShapeDtypeStruct((T, 16), jnp.float32),
                   jax.ShapeDtypeStruct((G_MAX + 1, 1), jnp.int32)],
    )(xf, Wg, bg.reshape(1, E))

    NW = 32
    meta_flat = meta.reshape(G_MAX + 1)
    dest_c = dest.reshape(2, NW, 8, 16)
    g0c = g0r.reshape(NW, 8, 16, 16)
    g1c = g1r.reshape(NW, 8, 16, 16)

    xg = _make_dispatch(T, EMB, NPAD)(xf, dest_c)

    yg = pl.pallas_call(
        functools.partial(_ffn_body, g_max=G_MAX),
        grid_spec=pltpu.PrefetchScalarGridSpec(
            num_scalar_prefetch=1,
            grid=(G_MAX,),
            in_specs=[
                pl.BlockSpec((TILE, EMB), lambda g, m: (g, 0)),
                pl.BlockSpec((1, EMB, HID), lambda g, m: (m[g], 0, 0)),
                pl.BlockSpec((1, 1, HID), lambda g, m: (m[g], 0, 0)),
                pl.BlockSpec((1, HID, EMB), lambda g, m: (m[g], 0, 0)),
                pl.BlockSpec((1, 1, EMB), lambda g, m: (m[g], 0, 0)),
            ],
            out_specs=pl.BlockSpec((TILE, EMB), lambda g, m: (g, 0)),
        ),
        out_shape=jax.ShapeDtypeStruct((NPAD, EMB), jnp.float32),
    )(meta_flat, xg, W1, b1.reshape(E, 1, HID),
      W2, b2.reshape(E, 1, EMB))

    out = _make_combine(T, EMB, NPAD)(yg, dest_c, g0c, g1c)
    return out.reshape(B, N, EMB)
